# baseline (speedup 1.0000x reference)
import functools

import jax
import jax.numpy as jnp
from jax import lax
from jax.experimental import pallas as pl
from jax.experimental.pallas import tpu as pltpu
from jax.experimental.pallas import tpu_sc as plsc

VOCAB = 100000
DIM = 64
BATCH = 4096
FIELDS = 26
TOTAL = BATCH * FIELDS

NUM_CORES = 2
NUM_SUBCORES = 16
NUM_WORKERS = NUM_CORES * NUM_SUBCORES
BPW = BATCH // NUM_WORKERS
IPW = BPW * FIELDS
TPITCH = 130

_mesh = plsc.VectorSubcoreMesh(core_axis_name="c", subcore_axis_name="s")


@functools.partial(
    pl.kernel,
    mesh=_mesh,
    out_type=jax.ShapeDtypeStruct((FIELDS, 8, NUM_WORKERS, 8, 128), jnp.float32),
    scratch_types=[
        pltpu.VMEM((IPW,), jnp.int32),
        pltpu.VMEM((FIELDS, BPW), jnp.int32),
        [pltpu.VMEM((BPW, DIM), jnp.float32) for _ in range(2)],
        [pltpu.VMEM((DIM, TPITCH), jnp.float32) for _ in range(2)],
        [pltpu.SemaphoreType.DMA for _ in range(2)],
        [pltpu.SemaphoreType.DMA for _ in range(2)],
    ],
    compiler_params=pltpu.CompilerParams(
        use_tc_tiling_on_sc=False, needs_layout_passes=False
    ),
)
def _gather_sc(table_hbm, idx_hbm, out_hbm, idx_v, idx_t, gbufs, tbufs,
               gsems, wsems):
    wid = lax.axis_index("s") * NUM_CORES + lax.axis_index("c")

    pltpu.sync_copy(idx_hbm.at[pl.ds(wid * IPW, IPW)], idx_v)

    @plsc.parallel_loop(0, FIELDS * (BPW // 16), 1, unroll=8)
    def _idx_transpose(t):
        f = t >> 3
        g = t & 7
        blv = g * 16 + lax.iota(jnp.int32, 16)
        vals = plsc.load_gather(idx_v, [blv * FIELDS + f])
        idx_t[f, pl.ds(g * 16, 16)] = vals * 2

    def issue_gather(f, p):
        return pltpu.async_copy(table_hbm.at[idx_t.at[f]], gbufs[p], gsems[p])

    def wait_gather(f, p):
        pltpu.make_async_copy(
            table_hbm.at[idx_t.at[f]], gbufs[p], gsems[p]
        ).wait()

    def transpose(p):
        gb, tb = gbufs[p], tbufs[p]

        @plsc.parallel_loop(0, BPW, 1, unroll=8)
        def _body(bl):
            for k in range(DIM // 16):
                row = gb[bl, pl.ds(k * 16, 16)]
                dv = k * 16 + lax.iota(jnp.int32, 16)
                blv = jnp.full((16,), bl, jnp.int32)
                plsc.store_scatter(tb, [dv, blv], row)

    def issue_wb(f, p):
        for dh in range(8):
            pltpu.async_copy(
                tbufs[p].at[pl.ds(dh * 8, 8), pl.ds(0, 128)],
                out_hbm.at[f, dh, wid],
                wsems[p],
            )

    def wait_wb(f, p):
        for dh in range(8):
            pltpu.make_async_copy(
                tbufs[p].at[pl.ds(dh * 8, 8), pl.ds(0, 128)],
                out_hbm.at[f, dh, wid],
                wsems[p],
            ).wait()

    def work(f, p):
        wait_gather(f, p)
        transpose(p)
        issue_wb(f, p)

    issue_gather(0, 0)
    issue_gather(1, 1)
    work(0, 0)
    issue_gather(2, 0)
    work(1, 1)
    issue_gather(3, 1)

    def loop_body(i, carry):
        f0 = 2 * i + 2
        wait_wb(f0 - 2, 0)
        work(f0, 0)
        issue_gather(f0 + 2, 0)
        wait_wb(f0 - 1, 1)
        work(f0 + 1, 1)
        issue_gather(f0 + 3, 1)
        return carry

    lax.fori_loop(0, (FIELDS - 4) // 2, loop_body, 0)

    wait_wb(FIELDS - 4, 0)
    work(FIELDS - 2, 0)
    wait_wb(FIELDS - 3, 1)
    work(FIELDS - 1, 1)
    wait_wb(FIELDS - 2, 0)
    wait_wb(FIELDS - 1, 1)


def kernel(input_indices, embedding_matrix):
    idx = input_indices.reshape(TOTAL).astype(jnp.int32)
    tpad = jnp.concatenate(
        [embedding_matrix, jnp.zeros((VOCAB, DIM), jnp.float32)], axis=1
    ).reshape(2 * VOCAB, DIM)
    out5 = _gather_sc(tpad, idx)
    return out5.transpose((2, 4, 0, 1, 3)).reshape(BATCH, FIELDS, DIM)


# speedup vs baseline: 1.0048x; 1.0048x over previous
import functools

import jax
import jax.numpy as jnp
from jax import lax
from jax.experimental import pallas as pl
from jax.experimental.pallas import tpu as pltpu
from jax.experimental.pallas import tpu_sc as plsc

VOCAB = 100000
DIM = 64
BATCH = 4096
FIELDS = 26
TOTAL = BATCH * FIELDS

NUM_CORES = 2
NUM_SUBCORES = 16
NUM_WORKERS = NUM_CORES * NUM_SUBCORES
BPW = BATCH // NUM_WORKERS
IPW = BPW * FIELDS
TPITCH = 130

_mesh = plsc.VectorSubcoreMesh(core_axis_name="c", subcore_axis_name="s")


@functools.partial(
    pl.kernel,
    mesh=_mesh,
    out_type=jax.ShapeDtypeStruct((FIELDS, 8, NUM_WORKERS, 8, 128), jnp.float32),
    scratch_types=[
        pltpu.VMEM((IPW,), jnp.int32),
        pltpu.VMEM((FIELDS, BPW), jnp.int32),
        [pltpu.VMEM((BPW, DIM), jnp.float32) for _ in range(2)],
        [pltpu.VMEM((DIM, TPITCH), jnp.float32) for _ in range(2)],
        [pltpu.SemaphoreType.DMA for _ in range(2)],
        [pltpu.SemaphoreType.DMA for _ in range(2)],
    ],
    compiler_params=pltpu.CompilerParams(
        use_tc_tiling_on_sc=False, needs_layout_passes=False
    ),
)
def _gather_sc(table_hbm, idx_hbm, out_hbm, idx_v, idx_t, gbufs, tbufs,
               gsems, wsems):
    wid = lax.axis_index("s") * NUM_CORES + lax.axis_index("c")

    pltpu.sync_copy(idx_hbm.at[pl.ds(wid * IPW, IPW)], idx_v)

    @plsc.parallel_loop(0, FIELDS * (BPW // 16), 1, unroll=8)
    def _idx_transpose(t):
        f = t >> 3
        g = t & 7
        blv = g * 16 + lax.iota(jnp.int32, 16)
        vals = plsc.load_gather(idx_v, [blv * FIELDS + f])
        idx_t[f, pl.ds(g * 16, 16)] = vals * 2

    def issue_gather(f, p):
        return pltpu.async_copy(table_hbm.at[idx_t.at[f]], gbufs[p], gsems[p])

    def wait_gather(f, p):
        pltpu.make_async_copy(
            table_hbm.at[idx_t.at[f]], gbufs[p], gsems[p]
        ).wait()

    def transpose(p):
        gb, tb = gbufs[p], tbufs[p]

        @plsc.parallel_loop(0, BPW, 1, unroll=4)
        def _body(bl):
            for k in range(DIM // 16):
                row = gb[bl, pl.ds(k * 16, 16)]
                dv = k * 16 + lax.iota(jnp.int32, 16)
                blv = jnp.full((16,), bl, jnp.int32)
                plsc.store_scatter(tb, [dv, blv], row)

    def issue_wb(f, p):
        for dh in range(8):
            pltpu.async_copy(
                tbufs[p].at[pl.ds(dh * 8, 8), pl.ds(0, 128)],
                out_hbm.at[f, dh, wid],
                wsems[p],
            )

    def wait_wb(f, p):
        for dh in range(8):
            pltpu.make_async_copy(
                tbufs[p].at[pl.ds(dh * 8, 8), pl.ds(0, 128)],
                out_hbm.at[f, dh, wid],
                wsems[p],
            ).wait()

    def work(f, p):
        wait_gather(f, p)
        transpose(p)
        issue_wb(f, p)

    issue_gather(0, 0)
    issue_gather(1, 1)
    work(0, 0)
    issue_gather(2, 0)
    work(1, 1)
    issue_gather(3, 1)

    def loop_body(i, carry):
        f0 = 2 * i + 2
        wait_wb(f0 - 2, 0)
        work(f0, 0)
        issue_gather(f0 + 2, 0)
        wait_wb(f0 - 1, 1)
        work(f0 + 1, 1)
        issue_gather(f0 + 3, 1)
        return carry

    lax.fori_loop(0, (FIELDS - 4) // 2, loop_body, 0)

    wait_wb(FIELDS - 4, 0)
    work(FIELDS - 2, 0)
    wait_wb(FIELDS - 3, 1)
    work(FIELDS - 1, 1)
    wait_wb(FIELDS - 2, 0)
    wait_wb(FIELDS - 1, 1)


def kernel(input_indices, embedding_matrix):
    idx = input_indices.reshape(TOTAL).astype(jnp.int32)
    tpad = jnp.concatenate(
        [embedding_matrix, jnp.zeros((VOCAB, DIM), jnp.float32)], axis=1
    ).reshape(2 * VOCAB, DIM)
    out5 = _gather_sc(tpad, idx)
    return out5.transpose((2, 4, 0, 1, 3)).reshape(BATCH, FIELDS, DIM)
